# baseline (speedup 1.0000x reference)
import functools

import jax
import jax.numpy as jnp
import numpy as np
from jax import lax
from jax.experimental import pallas as pl
from jax.experimental.pallas import tpu as pltpu
from jax.experimental.pallas import tpu_sc as plsc

B = 16384
L = 200
D = 64
NCLS = 1000
DW = D // 2

_NC = 2
_NS = 16
NW = _NC * _NS
BPW = B // NW
HALF = BPW // 2
LA = 96
LB = L - LA
NSLOT = 8

_PERM = np.concatenate([
    np.arange(0, 32, 2), np.arange(1, 32, 2),
    np.arange(32, 64, 2), np.arange(33, 64, 2)])

_CPK = 32768
_CPK4 = _CPK // 4
_NPK = -(-1000000 // _CPK)


def _tc_pack(tt):
    def body(x_ref, o_ref):
        xt = x_ref[...].T
        de = lax.broadcasted_iota(jnp.int32, (D, DW), 0)
        je = lax.broadcasted_iota(jnp.int32, (D, DW), 1)
        sel_e = (de == 2 * je).astype(jnp.float32)
        sel_o = (de == 2 * je + 1).astype(jnp.float32)
        a = jnp.dot(xt, sel_e, preferred_element_type=jnp.float32)
        b = jnp.dot(xt, sel_o, preferred_element_type=jnp.float32)
        au = lax.bitcast_convert_type(
            a.astype(jnp.bfloat16), jnp.uint16).astype(jnp.uint32)
        bu = lax.bitcast_convert_type(
            b.astype(jnp.bfloat16), jnp.uint16).astype(jnp.uint32)
        w = lax.bitcast_convert_type(au | (bu << 16), jnp.float32)
        o_ref[...] = jnp.concatenate(
            [w[k * _CPK4:(k + 1) * _CPK4] for k in range(4)], axis=1)

    return pl.pallas_call(
        body,
        grid=(_NPK,),
        in_specs=[pl.BlockSpec((D, _CPK), lambda i: (0, i))],
        out_specs=pl.BlockSpec((_CPK4, 4 * DW), lambda i: (i, 0)),
        out_shape=jax.ShapeDtypeStruct((_NPK * _CPK4, 4 * DW), jnp.float32),
    )(tt)


def _sc_embed_sum(idx_flat, table):
    mesh = plsc.VectorSubcoreMesh(core_axis_name="c", subcore_axis_name="s")

    @functools.partial(
        pl.kernel,
        mesh=mesh,
        out_type=jax.ShapeDtypeStruct((B, D), jnp.float32),
        scratch_types=[
            pltpu.VMEM((HALF * L,), jnp.int32),
            pltpu.VMEM((NSLOT, L, DW), jnp.float32),
            pltpu.VMEM((HALF, D), jnp.float32),
            [pltpu.SemaphoreType.DMA] * NSLOT,
        ],
        compiler_params=pltpu.CompilerParams(
            use_tc_tiling_on_sc=False, needs_layout_passes=False),
    )
    def run(idx_hbm, tab_hbm, out_hbm, idx_v, rows_v, out_v, sems):
        c = lax.axis_index("c")
        s = lax.axis_index("s")
        wid = s * _NC + c
        row0 = wid * BPW

        def issue(slot, r):
            pltpu.async_copy(
                tab_hbm.at[idx_v.at[pl.ds(L * r, LA)]],
                rows_v.at[slot, pl.ds(0, LA)], sems[slot])
            pltpu.async_copy(
                tab_hbm.at[idx_v.at[pl.ds(L * r + LA, LB)]],
                rows_v.at[slot, pl.ds(LA, LB)], sems[slot])

        def wait(slot):
            pltpu.make_async_copy(
                tab_hbm.at[idx_v.at[pl.ds(0, LA)]],
                rows_v.at[slot, pl.ds(0, LA)], sems[slot]).wait()
            pltpu.make_async_copy(
                tab_hbm.at[idx_v.at[pl.ds(0, LB)]],
                rows_v.at[slot, pl.ds(LA, LB)], sems[slot]).wait()

        def sum_row(slot, r):
            def body(i, accs):
                out = []
                for half in range(2):
                    w = plsc.bitcast(
                        rows_v[slot, i, pl.ds(half * 16, 16)], jnp.int32)
                    ev = plsc.bitcast(lax.shift_left(w, 16), jnp.float32)
                    od = plsc.bitcast(w & jnp.int32(-65536), jnp.float32)
                    out.append(accs[2 * half] + ev)
                    out.append(accs[2 * half + 1] + od)
                return tuple(out)

            accs = lax.fori_loop(
                0, L, body,
                tuple(jnp.zeros((16,), jnp.float32) for _ in range(4)),
                unroll=8)
            for q in range(4):
                out_v[r, pl.ds(q * 16, 16)] = accs[q]

        for h in range(2):
            hrow = row0 + h * HALF
            pltpu.sync_copy(idx_hbm.at[pl.ds(L * hrow, L * HALF)], idx_v)

            for k in range(NSLOT):
                issue(k, k)

            def group(g, _):
                for k in range(NSLOT):
                    r = g * NSLOT + k
                    wait(k)
                    sum_row(k, r)
                    nr = r + NSLOT

                    @pl.when(nr < HALF)
                    def _():
                        issue(k, nr)

                return 0

            lax.fori_loop(0, HALF // NSLOT, group, 0)
            pltpu.sync_copy(out_v, out_hbm.at[pl.ds(hrow, HALF)])

    return run(idx_flat, table)


_BM = 1024


def _tc_project_t(emb, bias2, w, pbt):
    def body(x_ref, b_ref, w_ref, pb_ref, o_ref):
        x = (x_ref[...] + b_ref[...]).astype(jnp.bfloat16)
        o_ref[...] = (
            lax.dot_general(w_ref[...].astype(jnp.bfloat16), x,
                            (((0,), (1,)), ((), ())),
                            preferred_element_type=jnp.float32)
            + pb_ref[...])

    return pl.pallas_call(
        body,
        grid=(B // _BM,),
        in_specs=[
            pl.BlockSpec((_BM, D), lambda i: (i, 0)),
            pl.BlockSpec((1, D), lambda i: (0, 0)),
            pl.BlockSpec((D, NCLS), lambda i: (0, 0)),
            pl.BlockSpec((NCLS, 1), lambda i: (0, 0)),
        ],
        out_specs=pl.BlockSpec((NCLS, _BM), lambda i: (0, i)),
        out_shape=jax.ShapeDtypeStruct((NCLS, B), jnp.float32),
    )(emb, bias2, w, pbt)


def kernel(inputs, embed_table, bias, proj_w, proj_b):
    blk = inputs // _CPK
    q = inputs % _CPK
    idx_lin = blk * _CPK + 4 * (q % _CPK4) + q // _CPK4
    idx_flat = idx_lin.reshape(B * L)
    tt_b, idx_b = lax.optimization_barrier((embed_table.T, idx_flat))
    table_pk = _tc_pack(tt_b).reshape(_NPK * _CPK, DW)
    emb = _sc_embed_sum(idx_b, table_pk)
    perm = jnp.asarray(_PERM)
    logits_t = _tc_project_t(emb, bias[perm].reshape(1, D), proj_w[perm, :],
                             proj_b.reshape(NCLS, 1))
    return logits_t.T


# speedup vs baseline: 1.0030x; 1.0030x over previous
import functools

import jax
import jax.numpy as jnp
import numpy as np
from jax import lax
from jax.experimental import pallas as pl
from jax.experimental.pallas import tpu as pltpu
from jax.experimental.pallas import tpu_sc as plsc

B = 16384
L = 200
D = 64
NCLS = 1000
DW = D // 2

_NC = 2
_NS = 16
NW = _NC * _NS
BPW = B // NW
HALF = BPW // 2
LA = 96
LB = L - LA
NSLOT = 8

_PERM = np.concatenate([
    np.arange(0, 32, 2), np.arange(1, 32, 2),
    np.arange(32, 64, 2), np.arange(33, 64, 2)])

_CPK = 32768
_CPK4 = _CPK // 4
_NPK = -(-1000000 // _CPK)


def _tc_pack(tt):
    def body(x_ref, o_ref):
        x = x_ref[...]
        de = lax.broadcasted_iota(jnp.int32, (D, DW), 0)
        je = lax.broadcasted_iota(jnp.int32, (D, DW), 1)
        sel_e = (de == 2 * je).astype(jnp.float32)
        sel_o = (de == 2 * je + 1).astype(jnp.float32)
        a = lax.dot_general(x, sel_e, (((0,), (0,)), ((), ())),
                            preferred_element_type=jnp.float32)
        b = lax.dot_general(x, sel_o, (((0,), (0,)), ((), ())),
                            preferred_element_type=jnp.float32)
        au = lax.bitcast_convert_type(
            a.astype(jnp.bfloat16), jnp.uint16).astype(jnp.uint32)
        bu = lax.bitcast_convert_type(
            b.astype(jnp.bfloat16), jnp.uint16).astype(jnp.uint32)
        w = lax.bitcast_convert_type(au | (bu << 16), jnp.float32)
        o_ref[...] = jnp.concatenate(
            [w[k * _CPK4:(k + 1) * _CPK4] for k in range(4)], axis=1)

    return pl.pallas_call(
        body,
        grid=(_NPK,),
        in_specs=[pl.BlockSpec((D, _CPK), lambda i: (0, i))],
        out_specs=pl.BlockSpec((_CPK4, 4 * DW), lambda i: (i, 0)),
        out_shape=jax.ShapeDtypeStruct((_NPK * _CPK4, 4 * DW), jnp.float32),
    )(tt)


def _sc_embed_sum(idx_flat, table):
    mesh = plsc.VectorSubcoreMesh(core_axis_name="c", subcore_axis_name="s")

    @functools.partial(
        pl.kernel,
        mesh=mesh,
        out_type=jax.ShapeDtypeStruct((B, D), jnp.float32),
        scratch_types=[
            pltpu.VMEM((HALF * L,), jnp.int32),
            pltpu.VMEM((NSLOT, L, DW), jnp.float32),
            pltpu.VMEM((HALF, D), jnp.float32),
            [pltpu.SemaphoreType.DMA] * NSLOT,
        ],
        compiler_params=pltpu.CompilerParams(
            use_tc_tiling_on_sc=False, needs_layout_passes=False),
    )
    def run(idx_hbm, tab_hbm, out_hbm, idx_v, rows_v, out_v, sems):
        c = lax.axis_index("c")
        s = lax.axis_index("s")
        wid = s * _NC + c
        row0 = wid * BPW

        def issue(slot, r):
            pltpu.async_copy(
                tab_hbm.at[idx_v.at[pl.ds(L * r, LA)]],
                rows_v.at[slot, pl.ds(0, LA)], sems[slot])
            pltpu.async_copy(
                tab_hbm.at[idx_v.at[pl.ds(L * r + LA, LB)]],
                rows_v.at[slot, pl.ds(LA, LB)], sems[slot])

        def wait(slot):
            pltpu.make_async_copy(
                tab_hbm.at[idx_v.at[pl.ds(0, LA)]],
                rows_v.at[slot, pl.ds(0, LA)], sems[slot]).wait()
            pltpu.make_async_copy(
                tab_hbm.at[idx_v.at[pl.ds(0, LB)]],
                rows_v.at[slot, pl.ds(LA, LB)], sems[slot]).wait()

        def sum_row(slot, r):
            def body(i, accs):
                out = []
                for half in range(2):
                    w = plsc.bitcast(
                        rows_v[slot, i, pl.ds(half * 16, 16)], jnp.int32)
                    ev = plsc.bitcast(lax.shift_left(w, 16), jnp.float32)
                    od = plsc.bitcast(w & jnp.int32(-65536), jnp.float32)
                    out.append(accs[2 * half] + ev)
                    out.append(accs[2 * half + 1] + od)
                return tuple(out)

            accs = lax.fori_loop(
                0, L, body,
                tuple(jnp.zeros((16,), jnp.float32) for _ in range(4)),
                unroll=8)
            for q in range(4):
                out_v[r, pl.ds(q * 16, 16)] = accs[q]

        for h in range(2):
            hrow = row0 + h * HALF
            pltpu.sync_copy(idx_hbm.at[pl.ds(L * hrow, L * HALF)], idx_v)

            for k in range(NSLOT):
                issue(k, k)

            def group(g, _):
                for k in range(NSLOT):
                    r = g * NSLOT + k
                    wait(k)
                    sum_row(k, r)
                    nr = r + NSLOT

                    @pl.when(nr < HALF)
                    def _():
                        issue(k, nr)

                return 0

            lax.fori_loop(0, HALF // NSLOT, group, 0)
            pltpu.sync_copy(out_v, out_hbm.at[pl.ds(hrow, HALF)])

    return run(idx_flat, table)


_BM = 1024


def _tc_project_t(emb, bias2, w, pbt):
    def body(x_ref, b_ref, w_ref, pb_ref, o_ref):
        x = (x_ref[...] + b_ref[...]).astype(jnp.bfloat16)
        o_ref[...] = (
            lax.dot_general(w_ref[...].astype(jnp.bfloat16), x,
                            (((0,), (1,)), ((), ())),
                            preferred_element_type=jnp.float32)
            + pb_ref[...])

    return pl.pallas_call(
        body,
        grid=(B // _BM,),
        in_specs=[
            pl.BlockSpec((_BM, D), lambda i: (i, 0)),
            pl.BlockSpec((1, D), lambda i: (0, 0)),
            pl.BlockSpec((D, NCLS), lambda i: (0, 0)),
            pl.BlockSpec((NCLS, 1), lambda i: (0, 0)),
        ],
        out_specs=pl.BlockSpec((NCLS, _BM), lambda i: (0, i)),
        out_shape=jax.ShapeDtypeStruct((NCLS, B), jnp.float32),
    )(emb, bias2, w, pbt)


def kernel(inputs, embed_table, bias, proj_w, proj_b):
    blk = inputs // _CPK
    q = inputs % _CPK
    idx_lin = blk * _CPK + 4 * (q % _CPK4) + q // _CPK4
    idx_flat = idx_lin.reshape(B * L)
    tt_b, idx_b = lax.optimization_barrier((embed_table.T, idx_flat))
    table_pk = _tc_pack(tt_b).reshape(_NPK * _CPK, DW)
    emb = _sc_embed_sum(idx_b, table_pk)
    perm = jnp.asarray(_PERM)
    logits_t = _tc_project_t(emb, bias[perm].reshape(1, D), proj_w[perm, :],
                             proj_b.reshape(NCLS, 1))
    return logits_t.T
